# Initial kernel scaffold; baseline (speedup 1.0000x reference)
import math

import jax
import jax.numpy as jnp
from jax.experimental import pallas as pl

_K = 16
_EPS = 1e-8
_T = 512


def _body(q_ref, o_ref, km_ref, vm_ref, g_ref, out_ref):
    q = q_ref[0, 0]
    km = km_ref[0]
    vm = vm_ref[0]
    hd = q.shape[-1]

    qn = q / jnp.maximum(
        jnp.sqrt(jnp.sum(q * q, axis=-1, keepdims=True)), _EPS)
    kn = km / jnp.maximum(
        jnp.sqrt(jnp.sum(km * km, axis=-1, keepdims=True)), _EPS)

    sims = jax.lax.dot_general(
        qn, kn, (((1,), (1,)), ((), ())),
        preferred_element_type=jnp.float32) * (1.0 / math.sqrt(hd))

    cur = sims
    thr = None
    for i in range(_K):
        m = jnp.max(cur, axis=-1, keepdims=True)
        if i < _K - 1:
            cur = jnp.where(cur >= m, -jnp.inf, cur)
        else:
            thr = m

    w = jnp.where(sims >= thr, sims, 0.0)
    wm = jax.lax.dot_general(
        w, vm, (((1,), (0,)), ((), ())),
        preferred_element_type=jnp.float32)

    g = jax.nn.sigmoid(g_ref[0, 0])
    out_ref[0, 0] = g * wm + (1.0 - g) * o_ref[0, 0]


def kernel(inputs, query, key, value, outputs, gate, key_memories, value_memories):
    del inputs, key, value
    B, H, S, HD = query.shape
    M = key_memories.shape[1]
    nt = S // _T
    grid = (H, B * nt)

    gate2 = gate.reshape(H, 1)

    out = pl.pallas_call(
        _body,
        grid=grid,
        in_specs=[
            pl.BlockSpec((1, 1, _T, HD), lambda h, t: (t // nt, h, t % nt, 0)),
            pl.BlockSpec((1, 1, _T, HD), lambda h, t: (t // nt, h, t % nt, 0)),
            pl.BlockSpec((1, M, HD), lambda h, t: (h, 0, 0)),
            pl.BlockSpec((1, M, HD), lambda h, t: (h, 0, 0)),
            pl.BlockSpec((1, 1), lambda h, t: (h, 0)),
        ],
        out_specs=pl.BlockSpec((1, 1, _T, HD), lambda h, t: (t // nt, h, t % nt, 0)),
        out_shape=jax.ShapeDtypeStruct((B, H, S, HD), jnp.float32),
    )(query, outputs, key_memories, value_memories, gate2)
    return out


# speedup vs baseline: 49.0202x; 49.0202x over previous
import math

import jax
import jax.numpy as jnp
from jax.experimental import pallas as pl

_K = 16
_EPS = 1e-8
_T = 512


def _body(q_ref, o_ref, km_ref, vm_ref, g_ref, out_ref):
    q = q_ref[0, 0]
    km = km_ref[0]
    vm = vm_ref[0]
    hd = q.shape[-1]

    qn = q / jnp.maximum(
        jnp.sqrt(jnp.sum(q * q, axis=-1, keepdims=True)), _EPS)
    kn = km / jnp.maximum(
        jnp.sqrt(jnp.sum(km * km, axis=-1, keepdims=True)), _EPS)

    sims = jax.lax.dot_general(
        qn, kn, (((1,), (1,)), ((), ())),
        preferred_element_type=jnp.float32) * (1.0 / math.sqrt(hd))

    cur = sims
    thr = None
    for i in range(_K):
        m = jnp.max(cur, axis=-1, keepdims=True)
        if i < _K - 1:
            cur = jnp.where(cur >= m, -jnp.inf, cur)
        else:
            thr = m

    w = jnp.where(sims >= thr, sims, 0.0)
    wm = jax.lax.dot_general(
        w, vm, (((1,), (0,)), ((), ())),
        preferred_element_type=jnp.float32)

    g = jax.nn.sigmoid(g_ref[0, 0, 0])
    out_ref[0, 0] = g * wm + (1.0 - g) * o_ref[0, 0]


def kernel(inputs, query, key, value, outputs, gate, key_memories, value_memories):
    del inputs, key, value
    B, H, S, HD = query.shape
    M = key_memories.shape[1]
    nt = S // _T
    grid = (H, B * nt)

    gate2 = gate.reshape(H, 1, 1)

    out = pl.pallas_call(
        _body,
        grid=grid,
        in_specs=[
            pl.BlockSpec((1, 1, _T, HD), lambda h, t: (t // nt, h, t % nt, 0)),
            pl.BlockSpec((1, 1, _T, HD), lambda h, t: (t // nt, h, t % nt, 0)),
            pl.BlockSpec((1, M, HD), lambda h, t: (h, 0, 0)),
            pl.BlockSpec((1, M, HD), lambda h, t: (h, 0, 0)),
            pl.BlockSpec((1, 1, 1), lambda h, t: (h, 0, 0)),
        ],
        out_specs=pl.BlockSpec((1, 1, _T, HD), lambda h, t: (t // nt, h, t % nt, 0)),
        out_shape=jax.ShapeDtypeStruct((B, H, S, HD), jnp.float32),
    )(query, outputs, key_memories, value_memories, gate2)
    return out
